# Initial kernel scaffold; baseline (speedup 1.0000x reference)
import jax
import jax.numpy as jnp
from jax.experimental import pallas as pl


def kernel(hidden_states, Wqkv, k_mem, v_mem, Wgate, bgate, Wo):
    raise NotImplementedError("write your pallas kernel here")


# speedup vs baseline: 1.9783x; 1.9783x over previous
import functools

import jax
import jax.numpy as jnp
from jax.experimental import pallas as pl

_B, _S, _D = 1, 2048, 2048
_H, _HKV, _DH = 16, 4, 128
_G = _H // _HKV
_CBS, _CSTR = 16, 8
_SEL, _NSEL = 16, 8
_WIN = 64
_NEG = -1e9
_NCMP = (_S - _CBS) // _CSTR + 1 + 1
_TQ = 256
_SCALE = _DH ** -0.5


def _matmul_kern(a_ref, b_ref, o_ref):
    o_ref[...] = jnp.dot(a_ref[...], b_ref[...],
                         preferred_element_type=jnp.float32)


def _proj(a, b, tile_m):
    m, k = a.shape
    _, n = b.shape
    return pl.pallas_call(
        _matmul_kern,
        grid=(m // tile_m,),
        in_specs=[pl.BlockSpec((tile_m, k), lambda i: (i, 0)),
                  pl.BlockSpec((k, n), lambda i: (0, 0))],
        out_specs=pl.BlockSpec((tile_m, n), lambda i: (i, 0)),
        out_shape=jax.ShapeDtypeStruct((m, n), jnp.float32),
    )(a, b)


def _pool_kern(kT_ref, v_ref, kmemT_ref, vmem_ref, ckmT_ref, cvm_ref):
    c0 = jax.lax.broadcasted_iota(jnp.int32, (_S, _NCMP), 0)
    j0 = jax.lax.broadcasted_iota(jnp.int32, (_S, _NCMP), 1)
    start = (j0 - 1) * _CSTR
    pT = jnp.where((j0 >= 1) & (c0 >= start) & (c0 < start + _CBS),
                   1.0 / _CBS, 0.0).astype(jnp.float32)
    pooledT = jnp.dot(kT_ref[...], pT, preferred_element_type=jnp.float32,
                      precision=jax.lax.Precision.HIGHEST)
    jcol = jax.lax.broadcasted_iota(jnp.int32, (_DH, _NCMP), 1)
    ckmT_ref[0] = jnp.where(jcol == 0, kmemT_ref[0], pooledT)

    jrow = jax.lax.broadcasted_iota(jnp.int32, (_NCMP, _S), 0)
    crow = jax.lax.broadcasted_iota(jnp.int32, (_NCMP, _S), 1)
    startr = (jrow - 1) * _CSTR
    p = jnp.where((jrow >= 1) & (crow >= startr) & (crow < startr + _CBS),
                  1.0 / _CBS, 0.0).astype(jnp.float32)
    pooled = jnp.dot(p, v_ref[...], preferred_element_type=jnp.float32,
                     precision=jax.lax.Precision.HIGHEST)
    jrow2 = jax.lax.broadcasted_iota(jnp.int32, (_NCMP, _DH), 0)
    cvm_ref[0] = jnp.where(jrow2 == 0, vmem_ref[0], pooled)


def _attn_kern(q_ref, kT_ref, v_ref, x_ref, ckmT_ref, cvm_ref,
               wgT_ref, bg_ref, o_ref):
    qi = pl.program_id(1)
    q0 = qi * _TQ

    pos = q0 + jax.lax.broadcasted_iota(jnp.int32, (_TQ, 1), 0)

    ccol = jax.lax.broadcasted_iota(jnp.int32, (_TQ, _NCMP), 1)
    blk_end = (ccol - 1) * _CSTR + _CBS - 1
    cmask = (ccol == 0) | (pos >= blk_end)

    ckmT = ckmT_ref[0]
    cvm = cvm_ref[0]

    impsum = jnp.zeros((_TQ, _NCMP), jnp.float32)
    co_list = []
    for g in range(_G):
        qg = q_ref[:, g * _DH:(g + 1) * _DH]
        csim = jnp.dot(qg, ckmT, preferred_element_type=jnp.float32) * _SCALE
        csim = jnp.where(cmask, csim, _NEG)
        cm = jnp.max(csim, axis=1, keepdims=True)
        ce = jnp.exp(csim - cm)
        cattn = ce / jnp.sum(ce, axis=1, keepdims=True)
        impsum = impsum + cattn
        co_list.append(jnp.dot(cattn, cvm, preferred_element_type=jnp.float32))

    nselblk = _S // _SEL
    mc = jax.lax.broadcasted_iota(jnp.int32, (_NCMP, nselblk), 0)
    mm = jax.lax.broadcasted_iota(jnp.int32, (_NCMP, nselblk), 1)
    selM = jnp.where((mc >= 1) & ((mc - 1) // 2 == mm), 1.0, 0.0)
    sel_imp = jnp.dot(impsum, selM, preferred_element_type=jnp.float32)

    lane = jax.lax.broadcasted_iota(jnp.int32, (_TQ, nselblk), 1)
    selmask = jnp.zeros((_TQ, nselblk), jnp.bool_)
    vals = sel_imp
    for _ in range(_NSEL):
        mx = jnp.max(vals, axis=1, keepdims=True)
        idx = jnp.min(jnp.where(vals == mx, lane, nselblk), axis=1,
                      keepdims=True)
        pick = lane == idx
        selmask = selmask | pick
        vals = jnp.where(pick, -1e30, vals)

    em = jax.lax.broadcasted_iota(jnp.int32, (nselblk, _S), 0)
    ec = jax.lax.broadcasted_iota(jnp.int32, (nselblk, _S), 1)
    expand = jnp.where(em == ec // _SEL, 1.0, 0.0)
    selx = jnp.dot(selmask.astype(jnp.float32), expand,
                   preferred_element_type=jnp.float32)
    kcol = jax.lax.broadcasted_iota(jnp.int32, (_TQ, _S), 1)
    causal = kcol <= pos
    own = (kcol // _SEL) == (pos // _SEL)
    fmask = ((selx > 0.5) | own) & causal
    wmask = causal & (pos - kcol < _WIN)

    graw = jnp.dot(x_ref[...], wgT_ref[0],
                   preferred_element_type=jnp.float32) + bg_ref[0]
    gsig = jax.nn.sigmoid(graw)

    kT = kT_ref[...]
    v = v_ref[...]
    for g in range(_G):
        qg = q_ref[:, g * _DH:(g + 1) * _DH]
        sim = jnp.dot(qg, kT, preferred_element_type=jnp.float32) * _SCALE

        fs = jnp.where(fmask, sim, _NEG)
        fm = jnp.max(fs, axis=1, keepdims=True)
        fe = jnp.exp(fs - fm)
        fattn = fe / jnp.sum(fe, axis=1, keepdims=True)
        fo = jnp.dot(fattn, v, preferred_element_type=jnp.float32)

        ss = jnp.where(wmask, sim, _NEG)
        sm = jnp.max(ss, axis=1, keepdims=True)
        se = jnp.exp(ss - sm)
        sattn = se / jnp.sum(se, axis=1, keepdims=True)
        so = jnp.dot(sattn, v, preferred_element_type=jnp.float32)

        g0 = gsig[:, 4 * g:4 * g + 1]
        g1 = gsig[:, 4 * g + 1:4 * g + 2]
        g2 = gsig[:, 4 * g + 2:4 * g + 3]
        o_ref[:, g * _DH:(g + 1) * _DH] = (
            g0 * co_list[g] + g1 * fo + g2 * so)


def kernel(hidden_states, Wqkv, k_mem, v_mem, Wgate, bgate, Wo):
    x = hidden_states.reshape(_S, _D)

    qkv = _proj(x, Wqkv.T, 256)
    qq = qkv[:, :_H * _DH]
    kT_all = qkv[:, _H * _DH:(_H + _HKV) * _DH].T
    vv = qkv[:, (_H + _HKV) * _DH:]

    k_memT = jnp.transpose(k_mem, (0, 2, 1))

    ckmT, cvm = pl.pallas_call(
        _pool_kern,
        grid=(_HKV,),
        in_specs=[pl.BlockSpec((_DH, _S), lambda h: (h, 0)),
                  pl.BlockSpec((_S, _DH), lambda h: (0, h)),
                  pl.BlockSpec((1, _DH, 1), lambda h: (h, 0, 0)),
                  pl.BlockSpec((1, 1, _DH), lambda h: (h, 0, 0))],
        out_specs=[pl.BlockSpec((1, _DH, _NCMP), lambda h: (h, 0, 0)),
                   pl.BlockSpec((1, _NCMP, _DH), lambda h: (h, 0, 0))],
        out_shape=[jax.ShapeDtypeStruct((_HKV, _DH, _NCMP), jnp.float32),
                   jax.ShapeDtypeStruct((_HKV, _NCMP, _DH), jnp.float32)],
    )(kT_all, vv, k_memT, v_mem)

    wg4 = Wgate.reshape(_HKV, _G, 3, _D)
    wg_pad = jnp.pad(wg4, ((0, 0), (0, 0), (0, 1), (0, 0)))
    wgT = wg_pad.reshape(_HKV, 4 * _G, _D).transpose(0, 2, 1)
    bg4 = bgate.reshape(_HKV, _G, 3)
    bg = jnp.pad(bg4, ((0, 0), (0, 0), (0, 1))).reshape(_HKV, 1, 4 * _G)

    attnout = pl.pallas_call(
        _attn_kern,
        grid=(_HKV, _S // _TQ),
        in_specs=[pl.BlockSpec((_TQ, _G * _DH), lambda h, i: (i, h)),
                  pl.BlockSpec((_DH, _S), lambda h, i: (h, 0)),
                  pl.BlockSpec((_S, _DH), lambda h, i: (0, h)),
                  pl.BlockSpec((_TQ, _D), lambda h, i: (i, 0)),
                  pl.BlockSpec((1, _DH, _NCMP), lambda h, i: (h, 0, 0)),
                  pl.BlockSpec((1, _NCMP, _DH), lambda h, i: (h, 0, 0)),
                  pl.BlockSpec((1, _D, 4 * _G), lambda h, i: (h, 0, 0)),
                  pl.BlockSpec((1, 1, 4 * _G), lambda h, i: (h, 0, 0))],
        out_specs=pl.BlockSpec((_TQ, _G * _DH), lambda h, i: (i, h)),
        out_shape=jax.ShapeDtypeStruct((_S, _H * _DH), jnp.float32),
    )(qq, kT_all, vv, x, ckmT, cvm, wgT, bg)

    y = _proj(attnout, Wo.T, 256)
    return y.reshape(_B, _S, _D)


# speedup vs baseline: 2.1979x; 1.1110x over previous
import functools

import jax
import jax.numpy as jnp
from jax.experimental import pallas as pl

_B, _S, _D = 1, 2048, 2048
_H, _HKV, _DH = 16, 4, 128
_G = _H // _HKV
_CBS, _CSTR = 16, 8
_SEL, _NSEL = 16, 8
_WIN = 64
_NEG = -1e9
_NCMP = (_S - _CBS) // _CSTR + 1 + 1
_TQ = 256
_SCALE = _DH ** -0.5


def _matmul_kern(a_ref, b_ref, o_ref):
    o_ref[...] = jnp.dot(a_ref[...], b_ref[...],
                         preferred_element_type=jnp.float32)


def _proj(a, b, tile_m):
    m, k = a.shape
    _, n = b.shape
    return pl.pallas_call(
        _matmul_kern,
        grid=(m // tile_m,),
        in_specs=[pl.BlockSpec((tile_m, k), lambda i: (i, 0)),
                  pl.BlockSpec((k, n), lambda i: (0, 0))],
        out_specs=pl.BlockSpec((tile_m, n), lambda i: (i, 0)),
        out_shape=jax.ShapeDtypeStruct((m, n), jnp.float32),
    )(a, b)


def _pool_kern(kT_ref, v_ref, kmemT_ref, vmem_ref, ckmT_ref, cvm_ref):
    c0 = jax.lax.broadcasted_iota(jnp.int32, (_S, _NCMP), 0)
    j0 = jax.lax.broadcasted_iota(jnp.int32, (_S, _NCMP), 1)
    start = (j0 - 1) * _CSTR
    pT = jnp.where((j0 >= 1) & (c0 >= start) & (c0 < start + _CBS),
                   1.0 / _CBS, 0.0).astype(jnp.float32)
    pooledT = jnp.dot(kT_ref[...], pT, preferred_element_type=jnp.float32,
                      precision=jax.lax.Precision.HIGHEST)
    jcol = jax.lax.broadcasted_iota(jnp.int32, (_DH, _NCMP), 1)
    ckmT_ref[0] = jnp.where(jcol == 0, kmemT_ref[0], pooledT)

    jrow = jax.lax.broadcasted_iota(jnp.int32, (_NCMP, _S), 0)
    crow = jax.lax.broadcasted_iota(jnp.int32, (_NCMP, _S), 1)
    startr = (jrow - 1) * _CSTR
    p = jnp.where((jrow >= 1) & (crow >= startr) & (crow < startr + _CBS),
                  1.0 / _CBS, 0.0).astype(jnp.float32)
    pooled = jnp.dot(p, v_ref[...], preferred_element_type=jnp.float32,
                     precision=jax.lax.Precision.HIGHEST)
    jrow2 = jax.lax.broadcasted_iota(jnp.int32, (_NCMP, _DH), 0)
    cvm_ref[0] = jnp.where(jrow2 == 0, vmem_ref[0], pooled)


def _attn_kern(q_ref, kT_ref, v_ref, x_ref, ckmT_ref, cvm_ref,
               wgT_ref, bg_ref, o_ref):
    qi = pl.program_id(1)
    q0 = qi * _TQ

    pos = q0 + jax.lax.broadcasted_iota(jnp.int32, (_TQ, 1), 0)

    ccol = jax.lax.broadcasted_iota(jnp.int32, (_TQ, _NCMP), 1)
    blk_end = (ccol - 1) * _CSTR + _CBS - 1
    cmask = (ccol == 0) | (pos >= blk_end)

    ckmT = ckmT_ref[0]
    cvm = cvm_ref[0]

    impsum = jnp.zeros((_TQ, _NCMP), jnp.float32)
    co_list = []
    for g in range(_G):
        qg = q_ref[:, g * _DH:(g + 1) * _DH]
        csim = jnp.dot(qg, ckmT, preferred_element_type=jnp.float32) * _SCALE
        csim = jnp.where(cmask, csim, _NEG)
        cm = jnp.max(csim, axis=1, keepdims=True)
        ce = jnp.exp(csim - cm)
        cattn = ce / jnp.sum(ce, axis=1, keepdims=True)
        impsum = impsum + cattn
        co_list.append(jnp.dot(cattn, cvm, preferred_element_type=jnp.float32))

    nselblk = _S // _SEL
    mc = jax.lax.broadcasted_iota(jnp.int32, (_NCMP, nselblk), 0)
    mm = jax.lax.broadcasted_iota(jnp.int32, (_NCMP, nselblk), 1)
    selM = jnp.where((mc >= 1) & ((mc - 1) // 2 == mm), 1.0, 0.0)
    sel_imp = jnp.dot(impsum, selM, preferred_element_type=jnp.float32)

    lane = jax.lax.broadcasted_iota(jnp.int32, (_TQ, nselblk), 1)
    selmask = jnp.zeros((_TQ, nselblk), jnp.bool_)
    vals = sel_imp
    for _ in range(_NSEL):
        mx = jnp.max(vals, axis=1, keepdims=True)
        idx = jnp.min(jnp.where(vals == mx, lane, nselblk), axis=1,
                      keepdims=True)
        pick = lane == idx
        selmask = selmask | pick
        vals = jnp.where(pick, -1e30, vals)

    graw = jnp.dot(x_ref[...], wgT_ref[0],
                   preferred_element_type=jnp.float32) + bg_ref[0]
    gsig = jax.nn.sigmoid(graw)

    selmask_f = selmask.astype(jnp.float32)
    qgs = [q_ref[:, g * _DH:(g + 1) * _DH] for g in range(_G)]

    _CK = 256
    em = jax.lax.broadcasted_iota(jnp.int32, (nselblk, _CK), 0)
    ecol = jax.lax.broadcasted_iota(jnp.int32, (nselblk, _CK), 1)
    ckcol = jax.lax.broadcasted_iota(jnp.int32, (_TQ, _CK), 1)

    def fbody(c, carry):
        ms, ls, accs = carry
        kT_c = kT_ref[:, pl.ds(c * _CK, _CK)]
        v_c = v_ref[pl.ds(c * _CK, _CK), :]
        col = c * _CK + ckcol
        expand_c = jnp.where(em == c * (_CK // _SEL) + ecol // _SEL, 1.0, 0.0)
        selx_c = jnp.dot(selmask_f, expand_c,
                         preferred_element_type=jnp.float32)
        fm_c = ((selx_c > 0.5) | ((col // _SEL) == (pos // _SEL))) \
            & (col <= pos)
        nms, nls, naccs = [], [], []
        for g in range(_G):
            sim = jnp.dot(qgs[g], kT_c,
                          preferred_element_type=jnp.float32) * _SCALE
            fs = jnp.where(fm_c, sim, _NEG)
            rm = jnp.max(fs, axis=1, keepdims=True)
            m_new = jnp.maximum(ms[g], rm)
            corr = jnp.exp(ms[g] - m_new)
            p = jnp.exp(fs - m_new)
            nms.append(m_new)
            nls.append(ls[g] * corr + jnp.sum(p, axis=1, keepdims=True))
            naccs.append(accs[g] * corr +
                         jnp.dot(p, v_c, preferred_element_type=jnp.float32))
        return tuple(nms), tuple(nls), tuple(naccs)

    init = (tuple(jnp.full((_TQ, 1), -1e30, jnp.float32) for _ in range(_G)),
            tuple(jnp.zeros((_TQ, 1), jnp.float32) for _ in range(_G)),
            tuple(jnp.zeros((_TQ, _DH), jnp.float32) for _ in range(_G)))
    fms, fls, faccs = jax.lax.fori_loop(0, qi + 1, fbody, init)

    band0 = jnp.maximum(qi - 1, 0) * _CK
    kT_b = kT_ref[:, pl.ds(band0, 2 * _CK)]
    v_b = v_ref[pl.ds(band0, 2 * _CK), :]
    bcol = band0 + jax.lax.broadcasted_iota(jnp.int32, (_TQ, 2 * _CK), 1)
    wmask = (bcol <= pos) & (pos - bcol < _WIN)

    for g in range(_G):
        ssim = jnp.dot(qgs[g], kT_b, preferred_element_type=jnp.float32) * _SCALE
        ss = jnp.where(wmask, ssim, _NEG)
        sm = jnp.max(ss, axis=1, keepdims=True)
        se = jnp.exp(ss - sm)
        so = jnp.dot(se, v_b, preferred_element_type=jnp.float32) \
            * (1.0 / jnp.sum(se, axis=1, keepdims=True))

        fo = faccs[g] * (1.0 / fls[g])

        g0 = gsig[:, 4 * g:4 * g + 1]
        g1 = gsig[:, 4 * g + 1:4 * g + 2]
        g2 = gsig[:, 4 * g + 2:4 * g + 3]
        o_ref[:, g * _DH:(g + 1) * _DH] = (
            g0 * co_list[g] + g1 * fo + g2 * so)


def kernel(hidden_states, Wqkv, k_mem, v_mem, Wgate, bgate, Wo):
    x = hidden_states.reshape(_S, _D)

    qkv = _proj(x, Wqkv.T, 256)
    qq = qkv[:, :_H * _DH]
    kT_all = qkv[:, _H * _DH:(_H + _HKV) * _DH].T
    vv = qkv[:, (_H + _HKV) * _DH:]

    k_memT = jnp.transpose(k_mem, (0, 2, 1))

    ckmT, cvm = pl.pallas_call(
        _pool_kern,
        grid=(_HKV,),
        in_specs=[pl.BlockSpec((_DH, _S), lambda h: (h, 0)),
                  pl.BlockSpec((_S, _DH), lambda h: (0, h)),
                  pl.BlockSpec((1, _DH, 1), lambda h: (h, 0, 0)),
                  pl.BlockSpec((1, 1, _DH), lambda h: (h, 0, 0))],
        out_specs=[pl.BlockSpec((1, _DH, _NCMP), lambda h: (h, 0, 0)),
                   pl.BlockSpec((1, _NCMP, _DH), lambda h: (h, 0, 0))],
        out_shape=[jax.ShapeDtypeStruct((_HKV, _DH, _NCMP), jnp.float32),
                   jax.ShapeDtypeStruct((_HKV, _NCMP, _DH), jnp.float32)],
    )(kT_all, vv, k_memT, v_mem)

    wg4 = Wgate.reshape(_HKV, _G, 3, _D)
    wg_pad = jnp.pad(wg4, ((0, 0), (0, 0), (0, 1), (0, 0)))
    wgT = wg_pad.reshape(_HKV, 4 * _G, _D).transpose(0, 2, 1)
    bg4 = bgate.reshape(_HKV, _G, 3)
    bg = jnp.pad(bg4, ((0, 0), (0, 0), (0, 1))).reshape(_HKV, 1, 4 * _G)

    attnout = pl.pallas_call(
        _attn_kern,
        grid=(_HKV, _S // _TQ),
        in_specs=[pl.BlockSpec((_TQ, _G * _DH), lambda h, i: (i, h)),
                  pl.BlockSpec((_DH, _S), lambda h, i: (h, 0)),
                  pl.BlockSpec((_S, _DH), lambda h, i: (0, h)),
                  pl.BlockSpec((_TQ, _D), lambda h, i: (i, 0)),
                  pl.BlockSpec((1, _DH, _NCMP), lambda h, i: (h, 0, 0)),
                  pl.BlockSpec((1, _NCMP, _DH), lambda h, i: (h, 0, 0)),
                  pl.BlockSpec((1, _D, 4 * _G), lambda h, i: (h, 0, 0)),
                  pl.BlockSpec((1, 1, 4 * _G), lambda h, i: (h, 0, 0))],
        out_specs=pl.BlockSpec((_TQ, _G * _DH), lambda h, i: (i, h)),
        out_shape=jax.ShapeDtypeStruct((_S, _H * _DH), jnp.float32),
    )(qq, kT_all, vv, x, ckmT, cvm, wgT, bg)

    y = _proj(attnout, Wo.T, 256)
    return y.reshape(_B, _S, _D)


# speedup vs baseline: 2.2718x; 1.0336x over previous
import functools

import jax
import jax.numpy as jnp
from jax.experimental import pallas as pl

_B, _S, _D = 1, 2048, 2048
_H, _HKV, _DH = 16, 4, 128
_G = _H // _HKV
_CBS, _CSTR = 16, 8
_SEL, _NSEL = 16, 8
_WIN = 64
_NEG = -1e9
_NCMP = (_S - _CBS) // _CSTR + 1 + 1
_TQ = 256
_SCALE = _DH ** -0.5


def _matmul_kern(a_ref, b_ref, o_ref):
    o_ref[...] = jnp.dot(a_ref[...], b_ref[...],
                         preferred_element_type=jnp.float32)


def _proj(a, b, tile_m):
    m, k = a.shape
    _, n = b.shape
    return pl.pallas_call(
        _matmul_kern,
        grid=(m // tile_m,),
        in_specs=[pl.BlockSpec((tile_m, k), lambda i: (i, 0)),
                  pl.BlockSpec((k, n), lambda i: (0, 0))],
        out_specs=pl.BlockSpec((tile_m, n), lambda i: (i, 0)),
        out_shape=jax.ShapeDtypeStruct((m, n), jnp.float32),
    )(a, b)


def _pool_kern(kT_ref, v_ref, kmemT_ref, vmem_ref, ckmT_ref, cvm_ref):
    c0 = jax.lax.broadcasted_iota(jnp.int32, (_S, _NCMP), 0)
    j0 = jax.lax.broadcasted_iota(jnp.int32, (_S, _NCMP), 1)
    start = (j0 - 1) * _CSTR
    pT = jnp.where((j0 >= 1) & (c0 >= start) & (c0 < start + _CBS),
                   1.0 / _CBS, 0.0).astype(jnp.float32)
    pooledT = jnp.dot(kT_ref[...], pT, preferred_element_type=jnp.float32,
                      precision=jax.lax.Precision.HIGHEST)
    jcol = jax.lax.broadcasted_iota(jnp.int32, (_DH, _NCMP), 1)
    ckmT_ref[0] = jnp.where(jcol == 0, kmemT_ref[0], pooledT)

    jrow = jax.lax.broadcasted_iota(jnp.int32, (_NCMP, _S), 0)
    crow = jax.lax.broadcasted_iota(jnp.int32, (_NCMP, _S), 1)
    startr = (jrow - 1) * _CSTR
    p = jnp.where((jrow >= 1) & (crow >= startr) & (crow < startr + _CBS),
                  1.0 / _CBS, 0.0).astype(jnp.float32)
    pooled = jnp.dot(p, v_ref[...], preferred_element_type=jnp.float32,
                     precision=jax.lax.Precision.HIGHEST)
    jrow2 = jax.lax.broadcasted_iota(jnp.int32, (_NCMP, _DH), 0)
    cvm_ref[0] = jnp.where(jrow2 == 0, vmem_ref[0], pooled)


def _attn_kern(q_ref, kT_ref, v_ref, x_ref, ckmT_ref, cvm_ref,
               wgT_ref, bg_ref, o_ref):
    qi = pl.program_id(1)
    q0 = qi * _TQ
    _Q4 = _G * _TQ

    pos4 = q0 + jax.lax.broadcasted_iota(jnp.int32, (_Q4, 1), 0) % _TQ
    Q4 = jnp.concatenate([q_ref[:, g * _DH:(g + 1) * _DH]
                          for g in range(_G)], axis=0)

    ccol = jax.lax.broadcasted_iota(jnp.int32, (_Q4, _NCMP), 1)
    blk_end = (ccol - 1) * _CSTR + _CBS - 1
    cmask = (ccol == 0) | (pos4 >= blk_end)
    ckmT = ckmT_ref[0]
    cvm = cvm_ref[0]
    csim = jnp.dot(Q4, ckmT, preferred_element_type=jnp.float32) * _SCALE
    csim = jnp.where(cmask, csim, _NEG)
    cm = jnp.max(csim, axis=1, keepdims=True)
    ce = jnp.exp(csim - cm)
    cattn = ce / jnp.sum(ce, axis=1, keepdims=True)
    co4 = jnp.dot(cattn, cvm, preferred_element_type=jnp.float32)

    impsum = ((cattn[0:_TQ] + cattn[_TQ:2 * _TQ]) + cattn[2 * _TQ:3 * _TQ]) \
        + cattn[3 * _TQ:4 * _TQ]

    nselblk = _S // _SEL
    mc = jax.lax.broadcasted_iota(jnp.int32, (_NCMP, nselblk), 0)
    mm = jax.lax.broadcasted_iota(jnp.int32, (_NCMP, nselblk), 1)
    selM = jnp.where((mc >= 1) & ((mc - 1) // 2 == mm), 1.0, 0.0)
    sel_imp = jnp.dot(impsum, selM, preferred_element_type=jnp.float32)

    selmask = jnp.zeros((_TQ, nselblk), jnp.bool_)
    vals = sel_imp
    for _ in range(_NSEL):
        mx = jnp.max(vals, axis=1, keepdims=True)
        pick = (vals == mx) & (mx > -1e30)
        selmask = selmask | pick
        vals = jnp.where(pick, -1e30, vals)

    graw = jnp.dot(x_ref[...], wgT_ref[0],
                   preferred_element_type=jnp.float32) + bg_ref[0]
    gsig = jax.nn.sigmoid(graw)

    selmask4 = jnp.concatenate([selmask.astype(jnp.float32)] * _G, axis=0)

    _CK = 256
    em = jax.lax.broadcasted_iota(jnp.int32, (nselblk, _CK), 0)
    ecol = jax.lax.broadcasted_iota(jnp.int32, (nselblk, _CK), 1)
    ckcol = jax.lax.broadcasted_iota(jnp.int32, (_Q4, _CK), 1)

    def fbody(c, carry):
        m, l, acc = carry
        kT_c = kT_ref[:, pl.ds(c * _CK, _CK)]
        v_c = v_ref[pl.ds(c * _CK, _CK), :]
        col = c * _CK + ckcol
        expand_c = jnp.where(em == c * (_CK // _SEL) + ecol // _SEL, 1.0, 0.0)
        selx_c = jnp.dot(selmask4, expand_c, preferred_element_type=jnp.float32)
        fm_c = ((selx_c > 0.5) | ((col // _SEL) == (pos4 // _SEL))) \
            & (col <= pos4)
        sim = jnp.dot(Q4, kT_c, preferred_element_type=jnp.float32) * _SCALE
        fs = jnp.where(fm_c, sim, _NEG)
        rm = jnp.max(fs, axis=1, keepdims=True)
        m_new = jnp.maximum(m, rm)
        corr = jnp.exp(m - m_new)
        p = jnp.exp(fs - m_new)
        l_new = l * corr + jnp.sum(p, axis=1, keepdims=True)
        acc_new = acc * corr + jnp.dot(p, v_c,
                                       preferred_element_type=jnp.float32)
        return m_new, l_new, acc_new

    init = (jnp.full((_Q4, 1), -1e30, jnp.float32),
            jnp.zeros((_Q4, 1), jnp.float32),
            jnp.zeros((_Q4, _DH), jnp.float32))
    fm_, fl_, facc = jax.lax.fori_loop(0, qi + 1, fbody, init)
    fo4 = facc * (1.0 / fl_)

    band0 = jnp.maximum(qi - 1, 0) * _CK
    kT_b = kT_ref[:, pl.ds(band0, 2 * _CK)]
    v_b = v_ref[pl.ds(band0, 2 * _CK), :]
    bcol = band0 + jax.lax.broadcasted_iota(jnp.int32, (_Q4, 2 * _CK), 1)
    wmask = (bcol <= pos4) & (pos4 - bcol < _WIN)
    ssim = jnp.dot(Q4, kT_b, preferred_element_type=jnp.float32) * _SCALE
    ss = jnp.where(wmask, ssim, _NEG)
    sm = jnp.max(ss, axis=1, keepdims=True)
    se = jnp.exp(ss - sm)
    so4 = jnp.dot(se, v_b, preferred_element_type=jnp.float32) \
        * (1.0 / jnp.sum(se, axis=1, keepdims=True))

    for g in range(_G):
        g0 = gsig[:, 4 * g:4 * g + 1]
        g1 = gsig[:, 4 * g + 1:4 * g + 2]
        g2 = gsig[:, 4 * g + 2:4 * g + 3]
        sl = slice(g * _TQ, (g + 1) * _TQ)
        o_ref[:, g * _DH:(g + 1) * _DH] = (
            g0 * co4[sl] + g1 * fo4[sl] + g2 * so4[sl])


def kernel(hidden_states, Wqkv, k_mem, v_mem, Wgate, bgate, Wo):
    x = hidden_states.reshape(_S, _D)

    qkv = _proj(x, Wqkv.T, 256)
    qq = qkv[:, :_H * _DH]
    kT_all = qkv[:, _H * _DH:(_H + _HKV) * _DH].T
    vv = qkv[:, (_H + _HKV) * _DH:]

    k_memT = jnp.transpose(k_mem, (0, 2, 1))

    ckmT, cvm = pl.pallas_call(
        _pool_kern,
        grid=(_HKV,),
        in_specs=[pl.BlockSpec((_DH, _S), lambda h: (h, 0)),
                  pl.BlockSpec((_S, _DH), lambda h: (0, h)),
                  pl.BlockSpec((1, _DH, 1), lambda h: (h, 0, 0)),
                  pl.BlockSpec((1, 1, _DH), lambda h: (h, 0, 0))],
        out_specs=[pl.BlockSpec((1, _DH, _NCMP), lambda h: (h, 0, 0)),
                   pl.BlockSpec((1, _NCMP, _DH), lambda h: (h, 0, 0))],
        out_shape=[jax.ShapeDtypeStruct((_HKV, _DH, _NCMP), jnp.float32),
                   jax.ShapeDtypeStruct((_HKV, _NCMP, _DH), jnp.float32)],
    )(kT_all, vv, k_memT, v_mem)

    wg4 = Wgate.reshape(_HKV, _G, 3, _D)
    wg_pad = jnp.pad(wg4, ((0, 0), (0, 0), (0, 1), (0, 0)))
    wgT = wg_pad.reshape(_HKV, 4 * _G, _D).transpose(0, 2, 1)
    bg4 = bgate.reshape(_HKV, _G, 3)
    bg = jnp.pad(bg4, ((0, 0), (0, 0), (0, 1))).reshape(_HKV, 1, 4 * _G)

    attnout = pl.pallas_call(
        _attn_kern,
        grid=(_HKV, _S // _TQ),
        in_specs=[pl.BlockSpec((_TQ, _G * _DH), lambda h, i: (i, h)),
                  pl.BlockSpec((_DH, _S), lambda h, i: (h, 0)),
                  pl.BlockSpec((_S, _DH), lambda h, i: (0, h)),
                  pl.BlockSpec((_TQ, _D), lambda h, i: (i, 0)),
                  pl.BlockSpec((1, _DH, _NCMP), lambda h, i: (h, 0, 0)),
                  pl.BlockSpec((1, _NCMP, _DH), lambda h, i: (h, 0, 0)),
                  pl.BlockSpec((1, _D, 4 * _G), lambda h, i: (h, 0, 0)),
                  pl.BlockSpec((1, 1, 4 * _G), lambda h, i: (h, 0, 0))],
        out_specs=pl.BlockSpec((_TQ, _G * _DH), lambda h, i: (i, h)),
        out_shape=jax.ShapeDtypeStruct((_S, _H * _DH), jnp.float32),
    )(qq, kT_all, vv, x, ckmT, cvm, wgT, bg)

    y = _proj(attnout, Wo.T, 256)
    return y.reshape(_B, _S, _D)
